# baseline (speedup 1.0000x reference)
import functools

import jax
import jax.numpy as jnp
from jax import lax
from jax.experimental import pallas as pl
from jax.experimental.pallas import tpu as pltpu
from jax.experimental.pallas import tpu_sc as plsc

_B, _L = 4096, 200
_NC, _NS = 2, 16
_NW = _NC * _NS
_ROWS_W = _B // _NW

_mesh = plsc.VectorSubcoreMesh(core_axis_name="c", subcore_axis_name="s")


@functools.partial(
    pl.kernel,
    mesh=_mesh,
    out_type=jax.ShapeDtypeStruct((_B, _L), jnp.float32),
    scratch_types=[
        pltpu.VMEM((_ROWS_W, _L), jnp.int32),
        pltpu.VMEM((_ROWS_W, _L), jnp.float32),
        pltpu.SemaphoreType.DMA,
    ],
)
def _lookup(idx_hbm, table_hbm, out_hbm, idx_v, rows_v, sem):
    wid = lax.axis_index("s") * _NC + lax.axis_index("c")
    r0 = wid * _ROWS_W
    pltpu.sync_copy(idx_hbm.at[pl.ds(r0, _ROWS_W), :], idx_v)

    _BATCH = 8

    def _step(i, _):
        descs = []
        for b in range(_BATCH):
            r = i * _BATCH + b
            descs.append(
                pltpu.async_copy(
                    table_hbm.at[idx_v.at[r, pl.ds(0, 128)]],
                    rows_v.at[r, pl.ds(0, 128)],
                    sem,
                )
            )
            descs.append(
                pltpu.async_copy(
                    table_hbm.at[idx_v.at[r, pl.ds(128, 72)]],
                    rows_v.at[r, pl.ds(128, 72)],
                    sem,
                )
            )
        for d in descs:
            d.wait()
        return 0

    lax.fori_loop(0, _ROWS_W // _BATCH, _step, 0)
    pltpu.sync_copy(rows_v, out_hbm.at[pl.ds(r0, _ROWS_W), :])


def kernel(indices, table):
    return _lookup(indices, table)


# speedup vs baseline: 1.0188x; 1.0188x over previous
import functools

import jax
import jax.numpy as jnp
from jax import lax
from jax.experimental import pallas as pl
from jax.experimental.pallas import tpu as pltpu
from jax.experimental.pallas import tpu_sc as plsc

_B, _L = 4096, 200
_NC, _NS = 2, 16
_NW = _NC * _NS
_ROWS_W = _B // _NW

_mesh = plsc.VectorSubcoreMesh(core_axis_name="c", subcore_axis_name="s")


@functools.partial(
    pl.kernel,
    mesh=_mesh,
    out_type=jax.ShapeDtypeStruct((_B, _L), jnp.float32),
    scratch_types=[
        pltpu.VMEM((_ROWS_W, _L), jnp.int32),
        pltpu.VMEM((_ROWS_W, _L), jnp.float32),
        pltpu.SemaphoreType.DMA,
    ],
)
def _lookup(idx_hbm, table_hbm, out_hbm, idx_v, rows_v, sem):
    wid = lax.axis_index("s") * _NC + lax.axis_index("c")
    r0 = wid * _ROWS_W
    pltpu.sync_copy(idx_hbm.at[pl.ds(r0, _ROWS_W), :], idx_v)

    _BATCH = 8
    _NB = _ROWS_W // _BATCH

    def _fire(i):
        for b in range(_BATCH):
            r = i * _BATCH + b
            pltpu.async_copy(
                table_hbm.at[idx_v.at[r, pl.ds(0, 128)]],
                rows_v.at[r, pl.ds(0, 128)],
                sem,
            )
            pltpu.async_copy(
                table_hbm.at[idx_v.at[r, pl.ds(128, 72)]],
                rows_v.at[r, pl.ds(128, 72)],
                sem,
            )

    def _drain(i):
        for b in range(_BATCH):
            r = i * _BATCH + b
            pltpu.make_async_copy(
                table_hbm.at[idx_v.at[r, pl.ds(0, 128)]],
                rows_v.at[r, pl.ds(0, 128)],
                sem,
            ).wait()
            pltpu.make_async_copy(
                table_hbm.at[idx_v.at[r, pl.ds(128, 72)]],
                rows_v.at[r, pl.ds(128, 72)],
                sem,
            ).wait()

    _fire(0)

    def _step(i, _):
        _fire(i + 1)
        _drain(i)
        return 0

    lax.fori_loop(0, _NB - 1, _step, 0)
    _drain(_NB - 1)
    pltpu.sync_copy(rows_v, out_hbm.at[pl.ds(r0, _ROWS_W), :])


def kernel(indices, table):
    return _lookup(indices, table)


# speedup vs baseline: 1.0191x; 1.0003x over previous
import functools

import jax
import jax.numpy as jnp
from jax import lax
from jax.experimental import pallas as pl
from jax.experimental.pallas import tpu as pltpu
from jax.experimental.pallas import tpu_sc as plsc

_B, _L = 4096, 200
_NC, _NS = 2, 16
_NW = _NC * _NS
_ROWS_W = _B // _NW

_mesh = plsc.VectorSubcoreMesh(core_axis_name="c", subcore_axis_name="s")


@functools.partial(
    pl.kernel,
    mesh=_mesh,
    out_type=jax.ShapeDtypeStruct((_B, _L), jnp.float32),
    compiler_params=pltpu.CompilerParams(use_tc_tiling_on_sc=True),
    scratch_types=[
        pltpu.VMEM((_ROWS_W, _L), jnp.int32),
        pltpu.VMEM((_ROWS_W, _L), jnp.float32),
        pltpu.SemaphoreType.DMA,
    ],
)
def _lookup(idx_hbm, table_hbm, out_hbm, idx_v, rows_v, sem):
    wid = lax.axis_index("s") * _NC + lax.axis_index("c")
    r0 = wid * _ROWS_W
    pltpu.sync_copy(idx_hbm.at[pl.ds(r0, _ROWS_W), :], idx_v)

    _BATCH = 8
    _NB = _ROWS_W // _BATCH

    def _fire(i):
        for b in range(_BATCH):
            r = i * _BATCH + b
            pltpu.async_copy(
                table_hbm.at[idx_v.at[r, pl.ds(0, 128)]],
                rows_v.at[r, pl.ds(0, 128)],
                sem,
            )
            pltpu.async_copy(
                table_hbm.at[idx_v.at[r, pl.ds(128, 72)]],
                rows_v.at[r, pl.ds(128, 72)],
                sem,
            )

    def _drain(i):
        for b in range(_BATCH):
            r = i * _BATCH + b
            pltpu.make_async_copy(
                table_hbm.at[idx_v.at[r, pl.ds(0, 128)]],
                rows_v.at[r, pl.ds(0, 128)],
                sem,
            ).wait()
            pltpu.make_async_copy(
                table_hbm.at[idx_v.at[r, pl.ds(128, 72)]],
                rows_v.at[r, pl.ds(128, 72)],
                sem,
            ).wait()

    _fire(0)

    def _step(i, _):
        _fire(i + 1)
        _drain(i)
        return 0

    lax.fori_loop(0, _NB - 1, _step, 0)
    _drain(_NB - 1)
    pltpu.sync_copy(rows_v, out_hbm.at[pl.ds(r0, _ROWS_W), :])


def kernel(indices, table):
    return _lookup(indices, table)


# speedup vs baseline: 1.1744x; 1.1524x over previous
import functools

import jax
import jax.numpy as jnp
from jax import lax
from jax.experimental import pallas as pl
from jax.experimental.pallas import tpu as pltpu
from jax.experimental.pallas import tpu_sc as plsc

_B, _L = 4096, 200
_NC, _NS = 2, 16
_NW = _NC * _NS
_COLS_W = _B // _NW

_mesh = plsc.VectorSubcoreMesh(core_axis_name="c", subcore_axis_name="s")


@functools.partial(
    pl.kernel,
    mesh=_mesh,
    out_type=jax.ShapeDtypeStruct((_L, _B), jnp.float32),
    compiler_params=pltpu.CompilerParams(use_tc_tiling_on_sc=True),
    scratch_types=[
        pltpu.VMEM((_L, _COLS_W), jnp.int32),
        pltpu.VMEM((_L, _COLS_W), jnp.float32),
        pltpu.SemaphoreType.DMA,
    ],
)
def _lookup(idx_hbm, table_hbm, out_hbm, idx_v, rows_v, sem):
    wid = lax.axis_index("s") * _NC + lax.axis_index("c")
    c0 = wid * _COLS_W
    pltpu.sync_copy(idx_hbm.at[:, pl.ds(c0, _COLS_W)], idx_v)

    _BATCH = 8
    _NB = _L // _BATCH

    def _fire(i):
        for b in range(_BATCH):
            r = i * _BATCH + b
            pltpu.async_copy(table_hbm.at[idx_v.at[r]], rows_v.at[r], sem)

    def _drain(i):
        for b in range(_BATCH):
            r = i * _BATCH + b
            pltpu.make_async_copy(
                table_hbm.at[idx_v.at[r]], rows_v.at[r], sem
            ).wait()

    _fire(0)

    def _step(i, _):
        _fire(i + 1)
        _drain(i)
        return 0

    lax.fori_loop(0, _NB - 1, _step, 0)
    _drain(_NB - 1)
    pltpu.sync_copy(rows_v, out_hbm.at[:, pl.ds(c0, _COLS_W)])


def kernel(indices, table):
    out_t = _lookup(indices.T, table)
    return out_t.T


# speedup vs baseline: 1.1817x; 1.0063x over previous
import functools

import jax
import jax.numpy as jnp
from jax import lax
from jax.experimental import pallas as pl
from jax.experimental.pallas import tpu as pltpu
from jax.experimental.pallas import tpu_sc as plsc

_B, _L = 4096, 200
_NC, _NS = 2, 16
_NW = _NC * _NS
_COLS_W = _B // _NW

_mesh = plsc.VectorSubcoreMesh(core_axis_name="c", subcore_axis_name="s")


@functools.partial(
    pl.kernel,
    mesh=_mesh,
    out_type=jax.ShapeDtypeStruct((_L, _B), jnp.float32),
    compiler_params=pltpu.CompilerParams(use_tc_tiling_on_sc=True),
    scratch_types=[
        pltpu.VMEM((_L, _COLS_W), jnp.int32),
        pltpu.VMEM((_L, _COLS_W), jnp.float32),
        pltpu.SemaphoreType.DMA,
    ],
)
def _lookup(idx_hbm, table_hbm, out_hbm, idx_v, rows_v, sem):
    wid = lax.axis_index("s") * _NC + lax.axis_index("c")
    c0 = wid * _COLS_W
    pltpu.sync_copy(idx_hbm.at[:, pl.ds(c0, _COLS_W)], idx_v)
    idx_r = idx_v.reshape(25, 8 * _COLS_W)
    rows_r = rows_v.reshape(25, 8 * _COLS_W)

    _BATCH = 1
    _NB = 25

    def _fire(i):
        pltpu.async_copy(table_hbm.at[idx_r.at[i]], rows_r.at[i], sem)

    def _drain(i):
        pltpu.make_async_copy(
            table_hbm.at[idx_r.at[i]], rows_r.at[i], sem
        ).wait()

    _fire(0)

    def _step(i, _):
        _fire(i + 1)
        _drain(i)
        return 0

    lax.fori_loop(0, _NB - 1, _step, 0)
    _drain(_NB - 1)
    pltpu.sync_copy(rows_v, out_hbm.at[:, pl.ds(c0, _COLS_W)])


def kernel(indices, table):
    out_t = _lookup(indices.T, table)
    return out_t.T


# speedup vs baseline: 1.3316x; 1.1268x over previous
import functools

import jax
import jax.numpy as jnp
from jax import lax
from jax.experimental import pallas as pl
from jax.experimental.pallas import tpu as pltpu
from jax.experimental.pallas import tpu_sc as plsc

_B, _L = 4096, 200
_NC, _NS = 2, 16
_NW = _NC * _NS
_COLS_W = _B // _NW

_mesh = plsc.VectorSubcoreMesh(core_axis_name="c", subcore_axis_name="s")


@functools.partial(
    pl.kernel,
    mesh=_mesh,
    out_type=jax.ShapeDtypeStruct((_L, _B), jnp.float32),
    compiler_params=pltpu.CompilerParams(use_tc_tiling_on_sc=True),
    scratch_types=[
        pltpu.VMEM((_L, _COLS_W), jnp.int32),
        pltpu.VMEM((_L, _COLS_W), jnp.float32),
        pltpu.SemaphoreType.DMA,
        pltpu.SemaphoreType.DMA,
    ],
)
def _lookup(idx_hbm, table_hbm, out_hbm, idx_v, rows_v, sem, sem2):
    wid = lax.axis_index("s") * _NC + lax.axis_index("c")
    c0 = wid * _COLS_W
    _NT = _L // 8
    _H1 = 13 * 8
    _H2 = _L - _H1
    idx_r = idx_v.reshape(_NT, 8 * _COLS_W)
    rows_r = rows_v.reshape(_NT, 8 * _COLS_W)

    pltpu.sync_copy(idx_hbm.at[pl.ds(0, _H1), pl.ds(c0, _COLS_W)],
                    idx_v.at[pl.ds(0, _H1), :])
    for k in range(13):
        pltpu.async_copy(table_hbm.at[idx_r.at[k]], rows_r.at[k], sem)
    pltpu.sync_copy(idx_hbm.at[pl.ds(_H1, _H2), pl.ds(c0, _COLS_W)],
                    idx_v.at[pl.ds(_H1, _H2), :])
    for k in range(13, _NT):
        pltpu.async_copy(table_hbm.at[idx_r.at[k]], rows_r.at[k], sem)

    for k in range(_NT):
        pltpu.make_async_copy(
            table_hbm.at[idx_r.at[k]], rows_r.at[k], sem
        ).wait()
        pltpu.async_copy(
            rows_v.at[pl.ds(8 * k, 8), :],
            out_hbm.at[pl.ds(8 * k, 8), pl.ds(c0, _COLS_W)],
            sem2,
        )
    for k in range(_NT):
        pltpu.make_async_copy(
            rows_v.at[pl.ds(8 * k, 8), :],
            out_hbm.at[pl.ds(8 * k, 8), pl.ds(c0, _COLS_W)],
            sem2,
        ).wait()


def kernel(indices, table):
    out_t = _lookup(indices.T, table)
    return out_t.T


# speedup vs baseline: 1.3859x; 1.0407x over previous
import functools

import jax
import jax.numpy as jnp
from jax import lax
from jax.experimental import pallas as pl
from jax.experimental.pallas import tpu as pltpu
from jax.experimental.pallas import tpu_sc as plsc

_B, _L = 4096, 200
_NC, _NS = 2, 16
_NW = _NC * _NS
_COLS_W = _B // _NW

_mesh = plsc.VectorSubcoreMesh(core_axis_name="c", subcore_axis_name="s")


@functools.partial(
    pl.kernel,
    mesh=_mesh,
    out_type=jax.ShapeDtypeStruct((_L, _B), jnp.float32),
    compiler_params=pltpu.CompilerParams(use_tc_tiling_on_sc=True),
    scratch_types=[
        pltpu.VMEM((_L, _COLS_W), jnp.int32),
        pltpu.VMEM((_L, _COLS_W), jnp.float32),
        pltpu.SemaphoreType.DMA,
        pltpu.SemaphoreType.DMA,
    ],
)
def _lookup(idx_hbm, table_hbm, out_hbm, idx_v, rows_v, sem, sem2):
    wid = lax.axis_index("s") * _NC + lax.axis_index("c")
    c0 = wid * _COLS_W
    _H1 = 104
    _H2 = _L - _H1
    _G = 100
    idx_r = idx_v.reshape(2, _G * _COLS_W)
    rows_r = rows_v.reshape(2, _G * _COLS_W)

    pltpu.sync_copy(idx_hbm.at[pl.ds(0, _H1), pl.ds(c0, _COLS_W)],
                    idx_v.at[pl.ds(0, _H1), :])
    pltpu.async_copy(table_hbm.at[idx_r.at[0]], rows_r.at[0], sem)
    pltpu.sync_copy(idx_hbm.at[pl.ds(_H1, _H2), pl.ds(c0, _COLS_W)],
                    idx_v.at[pl.ds(_H1, _H2), :])
    pltpu.async_copy(table_hbm.at[idx_r.at[1]], rows_r.at[1], sem)

    _S = ((0, 96), (96, 104))
    for g in range(2):
        pltpu.make_async_copy(
            table_hbm.at[idx_r.at[g]], rows_r.at[g], sem
        ).wait()
        s0, sn = _S[g]
        pltpu.async_copy(
            rows_v.at[pl.ds(s0, sn), :],
            out_hbm.at[pl.ds(s0, sn), pl.ds(c0, _COLS_W)],
            sem2,
        )
    for g in range(2):
        s0, sn = _S[g]
        pltpu.make_async_copy(
            rows_v.at[pl.ds(s0, sn), :],
            out_hbm.at[pl.ds(s0, sn), pl.ds(c0, _COLS_W)],
            sem2,
        ).wait()


def kernel(indices, table):
    out_t = _lookup(indices.T, table)
    return out_t.T
